# Initial kernel scaffold; baseline (speedup 1.0000x reference)
import jax
import jax.numpy as jnp
from jax.experimental import pallas as pl


def kernel(x, edge_index, edge_attr, W_mem, b_mem, g_mem, be_mem, W_q, W_k, W_v, W_o, W_eu, b_eu, g_eu, be_eu, g_en, be_en, W1, b1, W2, b2, g1, be1, g2, be2):
    raise NotImplementedError("write your pallas kernel here")


# speedup vs baseline: 1.4190x; 1.4190x over previous
import functools

import jax
import jax.numpy as jnp
import numpy as np
from jax import lax
from jax.experimental import pallas as pl
from jax.experimental.pallas import tpu as pltpu

N = 10000
E = 320000
D = 128
DE = 16
H = 8
DH = D // H
DFF = 512

TE = 1280
TN = 1000


def _ln(x, g, b, eps=1e-5):
    mu = jnp.mean(x, axis=-1, keepdims=True)
    var = jnp.mean((x - mu) ** 2, axis=-1, keepdims=True)
    return (x - mu) * jax.lax.rsqrt(var + eps) * g + b


def _pre_body(x_ref, wcat_ref, a_ref, b_ref, q_ref):
    r = jnp.dot(x_ref[...], wcat_ref[...], preferred_element_type=jnp.float32)
    a_ref[...] = r[:, :D]
    b_ref[...] = r[:, D:2 * D]
    q_ref[...] = r[:, 2 * D:]


def _pre_call(x, wcat):
    grid = (N // TN,)
    return pl.pallas_call(
        _pre_body,
        grid=grid,
        in_specs=[
            pl.BlockSpec((TN, D), lambda i: (i, 0)),
            pl.BlockSpec((D, 3 * D), lambda i: (0, 0)),
        ],
        out_specs=[
            pl.BlockSpec((TN, D), lambda i: (i, 0)),
            pl.BlockSpec((TN, D), lambda i: (i, 0)),
            pl.BlockSpec((TN, D), lambda i: (i, 0)),
        ],
        out_shape=[
            jax.ShapeDtypeStruct((N, D), jnp.float32),
            jax.ShapeDtypeStruct((N, D), jnp.float32),
            jax.ShapeDtypeStruct((N, D), jnp.float32),
        ],
    )(x, wcat)


def _edge_body(ab_ref, q_ref, ea_ref, wme_ref, bmem_ref, gmem_ref, bemem_ref,
               weu_ref, beu_ref, geu_ref, beeu_ref, gen_ref, been_ref,
               wk_ref, wv_ref, rt_ref, rb_ref,
               w_ref, p_ref, eau_ref):
    ab = ab_ref[...]
    ea = ea_ref[...]
    memp = ab + jnp.dot(ea, wme_ref[...], preferred_element_type=jnp.float32) + bmem_ref[...]
    mem = jax.nn.relu(_ln(memp, gmem_ref[...], bemem_ref[...]))
    delta = jax.nn.relu(_ln(
        jnp.dot(mem, weu_ref[...], preferred_element_type=jnp.float32) + beu_ref[...],
        geu_ref[...], beeu_ref[...]))
    eau_ref[...] = _ln(ea + delta, gen_ref[...], been_ref[...])
    k = jnp.dot(mem, wk_ref[...], preferred_element_type=jnp.float32)
    v = jnp.dot(mem, wv_ref[...], preferred_element_type=jnp.float32)
    logits = jnp.dot(q_ref[...] * k, rt_ref[...],
                     preferred_element_type=jnp.float32) * (1.0 / np.sqrt(DH))
    p = jnp.exp(logits)
    w_ref[...] = jnp.dot(p, rb_ref[...], preferred_element_type=jnp.float32) * v
    p_ref[...] = jnp.concatenate([p, jnp.zeros((TE, DE - H), jnp.float32)], axis=1)


def _edge_call(ab, q, ea, wme, bmem, gmem, bemem, weu, beu, geu, beeu,
               gen, been, wk, wv, rt, rb):
    grid = (E // TE,)
    edge_spec = pl.BlockSpec((TE, D), lambda i: (i, 0))
    ea_spec = pl.BlockSpec((TE, DE), lambda i: (i, 0))

    def _w(shape):
        return pl.BlockSpec(shape, lambda i: tuple(0 for _ in shape))

    return pl.pallas_call(
        _edge_body,
        grid=grid,
        in_specs=[
            edge_spec, edge_spec, ea_spec,
            _w((DE, D)), _w((1, D)), _w((1, D)), _w((1, D)),
            _w((D, DE)), _w((1, DE)), _w((1, DE)), _w((1, DE)),
            _w((1, DE)), _w((1, DE)),
            _w((D, D)), _w((D, D)), _w((D, H)), _w((H, D)),
        ],
        out_specs=[edge_spec, ea_spec, ea_spec],
        out_shape=[
            jax.ShapeDtypeStruct((E, D), jnp.float32),
            jax.ShapeDtypeStruct((E, DE), jnp.float32),
            jax.ShapeDtypeStruct((E, DE), jnp.float32),
        ],
    )(ab, q, ea, wme, bmem, gmem, bemem, weu, beu, geu, beeu, gen, been,
      wk, wv, rt, rb)


def _node_body(x_ref, s_ref, den_ref, rb_ref, wo_ref, w1_ref, b1_ref,
               w2_ref, b2_ref, g1_ref, be1_ref, g2_ref, be2_ref, out_ref):
    den = den_ref[...][:, :H]
    denb = jnp.dot(1.0 / (den + 1e-16), rb_ref[...],
                   preferred_element_type=jnp.float32)
    aggr = jnp.dot(s_ref[...] * denb, wo_ref[...],
                   preferred_element_type=jnp.float32)
    h1 = _ln(x_ref[...] + aggr, g1_ref[...], be1_ref[...])
    ffn = jnp.dot(
        jax.nn.relu(jnp.dot(h1, w1_ref[...], preferred_element_type=jnp.float32)
                    + b1_ref[...]),
        w2_ref[...], preferred_element_type=jnp.float32) + b2_ref[...]
    out_ref[...] = _ln(h1 + ffn, g2_ref[...], be2_ref[...])


def _node_call(x, s, den, rb, wo, w1, b1, w2, b2, g1, be1, g2, be2):
    grid = (N // TN,)

    def _w(shape):
        return pl.BlockSpec(shape, lambda i: tuple(0 for _ in shape))

    return pl.pallas_call(
        _node_body,
        grid=grid,
        in_specs=[
            pl.BlockSpec((TN, D), lambda i: (i, 0)),
            pl.BlockSpec((TN, D), lambda i: (i, 0)),
            pl.BlockSpec((TN, DE), lambda i: (i, 0)),
            _w((H, D)), _w((D, D)), _w((D, DFF)), _w((1, DFF)),
            _w((DFF, D)), _w((1, D)), _w((1, D)), _w((1, D)),
            _w((1, D)), _w((1, D)),
        ],
        out_specs=pl.BlockSpec((TN, D), lambda i: (i, 0)),
        out_shape=jax.ShapeDtypeStruct((N, D), jnp.float32),
    )(x, s, den, rb, wo, w1, b1, w2, b2, g1, be1, g2, be2)


def kernel(x, edge_index, edge_attr, W_mem, b_mem, g_mem, be_mem,
           W_q, W_k, W_v, W_o, W_eu, b_eu, g_eu, be_eu, g_en, be_en,
           W1, b1, W2, b2, g1, be1, g2, be2):
    src_idx = edge_index[0].astype(jnp.int32)
    dst_idx = edge_index[1].astype(jnp.int32)

    rt = np.zeros((D, H), np.float32)
    for h in range(H):
        rt[h * DH:(h + 1) * DH, h] = 1.0
    rt = jnp.asarray(rt)
    rb = jnp.asarray(rt.T.copy())

    wcat = jnp.concatenate([W_mem[:D], W_mem[D:2 * D], W_q], axis=1)
    tab_a, tab_b, tab_q = _pre_call(x, wcat)

    ab = tab_a[dst_idx] + tab_b[src_idx]
    q = tab_q[dst_idx]

    w, p, eau = _edge_call(
        ab, q, edge_attr, W_mem[2 * D:],
        b_mem.reshape(1, D), g_mem.reshape(1, D), be_mem.reshape(1, D),
        W_eu, b_eu.reshape(1, DE), g_eu.reshape(1, DE), be_eu.reshape(1, DE),
        g_en.reshape(1, DE), be_en.reshape(1, DE),
        W_k, W_v, rt, rb)

    s = jax.ops.segment_sum(w, dst_idx, num_segments=N)
    den = jax.ops.segment_sum(p, dst_idx, num_segments=N)

    out = _node_call(x, s, den, rb, W_o, W1, b1.reshape(1, DFF), W2,
                     b2.reshape(1, D), g1.reshape(1, D), be1.reshape(1, D),
                     g2.reshape(1, D), be2.reshape(1, D))
    return (out, eau)


# speedup vs baseline: 2.4379x; 1.7181x over previous
import functools

import jax
import jax.numpy as jnp
import numpy as np
from jax import lax
from jax.experimental import pallas as pl
from jax.experimental.pallas import tpu as pltpu
from jax.experimental.pallas import tpu_sc as plsc

N = 10000
E = 320000
D = 128
DE = 16
H = 8
DH = D // H
DFF = 512

TE = 1280
TN = 1000


def _ln(x, g, b, eps=1e-5):
    mu = jnp.mean(x, axis=-1, keepdims=True)
    var = jnp.mean((x - mu) ** 2, axis=-1, keepdims=True)
    return (x - mu) * jax.lax.rsqrt(var + eps) * g + b


def _pre_body(x_ref, wcat_ref, a_ref, b_ref, q_ref):
    r = jnp.dot(x_ref[...], wcat_ref[...], preferred_element_type=jnp.float32)
    a_ref[...] = r[:, :D]
    b_ref[...] = r[:, D:2 * D]
    q_ref[...] = r[:, 2 * D:]


def _pre_call(x, wcat):
    grid = (N // TN,)
    return pl.pallas_call(
        _pre_body,
        grid=grid,
        in_specs=[
            pl.BlockSpec((TN, D), lambda i: (i, 0)),
            pl.BlockSpec((D, 3 * D), lambda i: (0, 0)),
        ],
        out_specs=[
            pl.BlockSpec((TN, D), lambda i: (i, 0)),
            pl.BlockSpec((TN, D), lambda i: (i, 0)),
            pl.BlockSpec((TN, D), lambda i: (i, 0)),
        ],
        out_shape=[
            jax.ShapeDtypeStruct((N, D), jnp.float32),
            jax.ShapeDtypeStruct((N, D), jnp.float32),
            jax.ShapeDtypeStruct((N, D), jnp.float32),
        ],
    )(x, wcat)


NC = 2
NS = 16
NW = NC * NS
EPW = E // NW
CG = 200
_SC_MESH = dict(core_axis_name="c", subcore_axis_name="s")


def _gather_body(tab_a, tab_b, tab_q, dst_hbm, src_hbm, ab_hbm, q_hbm,
                 idxd, idxs, av, bv, qv, sem1, sem2, sem3):
    wid = lax.axis_index("s") * NC + lax.axis_index("c")
    base = wid * EPW

    def chunk(c, carry):
        off = base + c * CG
        pltpu.sync_copy(dst_hbm.at[pl.ds(off, CG)], idxd)
        pltpu.sync_copy(src_hbm.at[pl.ds(off, CG)], idxs)
        cp1 = pltpu.async_copy(tab_a.at[idxd], av, sem1)
        cp2 = pltpu.async_copy(tab_b.at[idxs], bv, sem2)
        cp3 = pltpu.async_copy(tab_q.at[idxd], qv, sem3)
        cp1.wait()
        cp2.wait()
        cp3.wait()

        def row(i, carry2):
            for j in range(D // 16):
                sl = pl.ds(j * 16, 16)
                av[i, sl] = av[i, sl] + bv[i, sl]
            return carry2

        lax.fori_loop(0, CG, row, 0, unroll=2)
        pltpu.sync_copy(av, ab_hbm.at[pl.ds(off, CG)])
        pltpu.sync_copy(qv, q_hbm.at[pl.ds(off, CG)])
        return carry

    lax.fori_loop(0, EPW // CG, chunk, 0)


def _gather_call(tab_a, tab_b, tab_q, dst_idx, src_idx):
    mesh = plsc.VectorSubcoreMesh(**_SC_MESH)
    f = functools.partial(
        pl.kernel, mesh=mesh,
        out_type=[
            jax.ShapeDtypeStruct((E, D), jnp.float32),
            jax.ShapeDtypeStruct((E, D), jnp.float32),
        ],
        scratch_types=[
            pltpu.VMEM((CG,), jnp.int32),
            pltpu.VMEM((CG,), jnp.int32),
            pltpu.VMEM((CG, D), jnp.float32),
            pltpu.VMEM((CG, D), jnp.float32),
            pltpu.VMEM((CG, D), jnp.float32),
            pltpu.SemaphoreType.DMA,
            pltpu.SemaphoreType.DMA,
            pltpu.SemaphoreType.DMA,
        ],
    )(_gather_body)
    return f(tab_a, tab_b, tab_q, dst_idx, src_idx)


def _edge_body(ab_ref, q_ref, ea_ref, wme_ref, bmem_ref, gmem_ref, bemem_ref,
               weu_ref, beu_ref, geu_ref, beeu_ref, gen_ref, been_ref,
               wk_ref, wv_ref, rt_ref, rb_ref,
               w_ref, p_ref, eau_ref):
    ab = ab_ref[...]
    ea = ea_ref[...]
    memp = ab + jnp.dot(ea, wme_ref[...], preferred_element_type=jnp.float32) + bmem_ref[...]
    mem = jax.nn.relu(_ln(memp, gmem_ref[...], bemem_ref[...]))
    delta = jax.nn.relu(_ln(
        jnp.dot(mem, weu_ref[...], preferred_element_type=jnp.float32) + beu_ref[...],
        geu_ref[...], beeu_ref[...]))
    eau_ref[...] = _ln(ea + delta, gen_ref[...], been_ref[...])
    k = jnp.dot(mem, wk_ref[...], preferred_element_type=jnp.float32)
    v = jnp.dot(mem, wv_ref[...], preferred_element_type=jnp.float32)
    logits = jnp.dot(q_ref[...] * k, rt_ref[...],
                     preferred_element_type=jnp.float32) * (1.0 / np.sqrt(DH))
    p = jnp.exp(logits)
    w_ref[...] = jnp.dot(p, rb_ref[...], preferred_element_type=jnp.float32) * v
    p_ref[...] = jnp.concatenate([p, jnp.zeros((TE, DE - H), jnp.float32)], axis=1)


def _edge_call(ab, q, ea, wme, bmem, gmem, bemem, weu, beu, geu, beeu,
               gen, been, wk, wv, rt, rb):
    grid = (E // TE,)
    edge_spec = pl.BlockSpec((TE, D), lambda i: (i, 0))
    ea_spec = pl.BlockSpec((TE, DE), lambda i: (i, 0))

    def _w(shape):
        return pl.BlockSpec(shape, lambda i: tuple(0 for _ in shape))

    return pl.pallas_call(
        _edge_body,
        grid=grid,
        in_specs=[
            edge_spec, edge_spec, ea_spec,
            _w((DE, D)), _w((1, D)), _w((1, D)), _w((1, D)),
            _w((D, DE)), _w((1, DE)), _w((1, DE)), _w((1, DE)),
            _w((1, DE)), _w((1, DE)),
            _w((D, D)), _w((D, D)), _w((D, H)), _w((H, D)),
        ],
        out_specs=[edge_spec, ea_spec, ea_spec],
        out_shape=[
            jax.ShapeDtypeStruct((E, D), jnp.float32),
            jax.ShapeDtypeStruct((E, DE), jnp.float32),
            jax.ShapeDtypeStruct((E, DE), jnp.float32),
        ],
    )(ab, q, ea, wme, bmem, gmem, bemem, weu, beu, geu, beeu, gen, been,
      wk, wv, rt, rb)


def _node_body(x_ref, s_ref, den_ref, rb_ref, wo_ref, w1_ref, b1_ref,
               w2_ref, b2_ref, g1_ref, be1_ref, g2_ref, be2_ref, out_ref):
    den = den_ref[...][:, :H]
    denb = jnp.dot(1.0 / (den + 1e-16), rb_ref[...],
                   preferred_element_type=jnp.float32)
    aggr = jnp.dot(s_ref[...] * denb, wo_ref[...],
                   preferred_element_type=jnp.float32)
    h1 = _ln(x_ref[...] + aggr, g1_ref[...], be1_ref[...])
    ffn = jnp.dot(
        jax.nn.relu(jnp.dot(h1, w1_ref[...], preferred_element_type=jnp.float32)
                    + b1_ref[...]),
        w2_ref[...], preferred_element_type=jnp.float32) + b2_ref[...]
    out_ref[...] = _ln(h1 + ffn, g2_ref[...], be2_ref[...])


def _node_call(x, s, den, rb, wo, w1, b1, w2, b2, g1, be1, g2, be2):
    grid = (N // TN,)

    def _w(shape):
        return pl.BlockSpec(shape, lambda i: tuple(0 for _ in shape))

    return pl.pallas_call(
        _node_body,
        grid=grid,
        in_specs=[
            pl.BlockSpec((TN, D), lambda i: (i, 0)),
            pl.BlockSpec((TN, D), lambda i: (i, 0)),
            pl.BlockSpec((TN, DE), lambda i: (i, 0)),
            _w((H, D)), _w((D, D)), _w((D, DFF)), _w((1, DFF)),
            _w((DFF, D)), _w((1, D)), _w((1, D)), _w((1, D)),
            _w((1, D)), _w((1, D)),
        ],
        out_specs=pl.BlockSpec((TN, D), lambda i: (i, 0)),
        out_shape=jax.ShapeDtypeStruct((N, D), jnp.float32),
    )(x, s, den, rb, wo, w1, b1, w2, b2, g1, be1, g2, be2)


def kernel(x, edge_index, edge_attr, W_mem, b_mem, g_mem, be_mem,
           W_q, W_k, W_v, W_o, W_eu, b_eu, g_eu, be_eu, g_en, be_en,
           W1, b1, W2, b2, g1, be1, g2, be2):
    src_idx = edge_index[0].astype(jnp.int32)
    dst_idx = edge_index[1].astype(jnp.int32)

    rt = np.zeros((D, H), np.float32)
    for h in range(H):
        rt[h * DH:(h + 1) * DH, h] = 1.0
    rt = jnp.asarray(rt)
    rb = jnp.asarray(rt.T.copy())

    wcat = jnp.concatenate([W_mem[:D], W_mem[D:2 * D], W_q], axis=1)
    tab_a, tab_b, tab_q = _pre_call(x, wcat)

    ab, q = _gather_call(tab_a, tab_b, tab_q, dst_idx, src_idx)

    w, p, eau = _edge_call(
        ab, q, edge_attr, W_mem[2 * D:],
        b_mem.reshape(1, D), g_mem.reshape(1, D), be_mem.reshape(1, D),
        W_eu, b_eu.reshape(1, DE), g_eu.reshape(1, DE), be_eu.reshape(1, DE),
        g_en.reshape(1, DE), be_en.reshape(1, DE),
        W_k, W_v, rt, rb)

    s = jax.ops.segment_sum(w, dst_idx, num_segments=N)
    den = jax.ops.segment_sum(p, dst_idx, num_segments=N)

    out = _node_call(x, s, den, rb, W_o, W1, b1.reshape(1, DFF), W2,
                     b2.reshape(1, D), g1.reshape(1, D), be1.reshape(1, D),
                     g2.reshape(1, D), be2.reshape(1, D))
    return (out, eau)
